# baseline (speedup 1.0000x reference)
import jax
import jax.numpy as jnp
from jax import lax
from jax.experimental import pallas as pl
from jax.experimental.pallas import tpu as pltpu
from jax.experimental.pallas import tpu_sc as plsc

N0 = 10000
N1 = 10000
E = 320000

NC = 2
NS = 16
NW = NC * NS
L = 16
CHUNK = 128
NCHUNKS = E // CHUNK
MAXC = -(-NCHUNKS // NW)
ZROWS = 125


def _worker_split(wid):
    base_n = NCHUNKS // NW
    cut = NW - (NCHUNKS - base_n * NW)
    n_w = base_n + jnp.where(wid >= cut, 1, 0)
    chunk0 = wid * base_n + jnp.maximum(wid - cut, 0)
    return n_w, chunk0, base_n + 1


def _sc_degrees(edge_sets, n_out):
    mesh = plsc.VectorSubcoreMesh(core_axis_name="c", subcore_axis_name="s")
    nsets = len(edge_sets)
    out_type = jax.ShapeDtypeStruct((nsets, NC, n_out, L), jnp.float32)
    scratch = (
        [pltpu.VMEM((4, 4, 2, CHUNK), jnp.int32),
         pltpu.VMEM((CHUNK, L), jnp.float32)]
        + [pltpu.VMEM_SHARED((n_out, L), jnp.float32) for _ in range(nsets)]
        + [pltpu.SemaphoreType.DMA for _ in range(8)]
    )

    def body(*refs):
        e_hs = refs[:nsets]
        z_h = refs[nsets]
        out_h = refs[nsets + 1]
        idxv, onesv = refs[nsets + 2:nsets + 4]
        daccs = refs[nsets + 4:nsets + 4 + nsets]
        isems = refs[nsets + 4 + nsets:nsets + 4 + nsets + 4]
        ssems = refs[nsets + 4 + nsets + 4:]
        core = lax.axis_index("c")
        sid = lax.axis_index("s")
        wid = core * NS + sid
        n_w, chunk0, maxn = _worker_split(wid)

        @pl.loop(0, CHUNK)
        def _(i):
            onesv[i, :] = jnp.ones((L,), jnp.float32)

        for e in range(nsets):
            for k in range(5):
                pltpu.make_async_copy(
                    z_h, daccs[e].at[pl.ds(sid * 625 + k * ZROWS, ZROWS)],
                    isems[k % 2]).start()
        for e in range(nsets):
            for k in range(5):
                pltpu.make_async_copy(
                    z_h, daccs[e].at[pl.ds(sid * 625 + k * ZROWS, ZROWS)],
                    isems[k % 2]).wait()
        plsc.subcore_barrier()

        for e in range(nsets):
            e_h, dacc = e_hs[e], daccs[e]

            def start_blk(g, s, e_h=e_h):
                pltpu.make_async_copy(e_h.at[pl.ds(chunk0 + 4 * g, 4)],
                                      idxv.at[s], isems[s]).start()

            def wait_blk(g, s, e_h=e_h):
                pltpu.make_async_copy(e_h.at[pl.ds(chunk0 + 4 * g, 4)],
                                      idxv.at[s], isems[s]).wait()

            for s in range(3):
                start_blk(s, s)

            @pl.loop(0, maxn + 17, step=16)
            def _(j):
                for b in range(16):
                    k = j + b
                    blk, win = (b // 4) % 4, b % 4

                    @pl.when(k < n_w)
                    def _():
                        if win == 0:
                            wait_blk(k // 4, blk)
                        pltpu.async_copy(onesv, dacc.at[idxv.at[blk, win, 0]],
                                         ssems[win], add=True)

                    @pl.when((k >= 3) & (k - 3 < n_w))
                    def _():
                        pltpu.make_async_copy(
                            onesv, dacc.at[idxv.at[blk, win, 0]],
                            ssems[(win + 1) % 4]).wait()

                    if win == 2:
                        @pl.when(k + 10 < n_w)
                        def _():
                            start_blk(k // 4 + 3, (blk + 3) % 4)

        plsc.subcore_barrier()

        for e in range(nsets):
            pltpu.sync_copy(daccs[e].at[pl.ds(sid * 625, 625)],
                            out_h.at[e, core, pl.ds(sid * 625, 625)])

    kfn = pl.kernel(body, mesh=mesh, scratch_types=scratch, out_type=out_type,
                    compiler_params=pltpu.CompilerParams(
                        use_tc_tiling_on_sc=False))
    return kfn(*edge_sets, jnp.zeros((ZROWS, L), jnp.float32))


def _sc_agg(table, edges, n_out):
    d = table.shape[1]
    mesh = plsc.VectorSubcoreMesh(core_axis_name="c", subcore_axis_name="s")

    out_type = jax.ShapeDtypeStruct((NC, n_out, d), jnp.float32)
    scratch = [
        pltpu.VMEM((6, 2, CHUNK), jnp.int32),
        pltpu.VMEM((CHUNK, d), jnp.float32),
        pltpu.VMEM((CHUNK, d), jnp.float32),
        pltpu.VMEM((CHUNK, d), jnp.float32),
        pltpu.VMEM_SHARED((n_out, d), jnp.float32),
    ] + [pltpu.SemaphoreType.DMA] * 12
    def body(table_h, e_h, z_h, out_h, idxv, rows0, rows1, rows2, acc, *sems):
        isems, gsems, ssems = sems[:6], sems[6:9], sems[9:12]
        rows = (rows0, rows1, rows2)
        core = lax.axis_index("c")
        sid = lax.axis_index("s")
        wid = core * NS + sid
        n_w, chunk0, maxn = _worker_split(wid)

        for k in range(5):
            pltpu.make_async_copy(
                z_h, acc.at[pl.ds(sid * 625 + k * ZROWS, ZROWS)],
                gsems[k % 2]).start()
        for k in range(5):
            pltpu.make_async_copy(
                z_h, acc.at[pl.ds(sid * 625 + k * ZROWS, ZROWS)],
                gsems[k % 2]).wait()
        plsc.subcore_barrier()

        def start_idx(k, s):
            pltpu.make_async_copy(e_h.at[chunk0 + k], idxv.at[s],
                                  isems[s]).start()

        def wait_idx(k, s):
            pltpu.make_async_copy(e_h.at[chunk0 + k], idxv.at[s],
                                  isems[s]).wait()

        def start_gather(s, b):
            pltpu.make_async_copy(table_h.at[idxv.at[s, 1]], rows[b],
                                  gsems[b]).start()

        def wait_gather(s, b):
            pltpu.make_async_copy(table_h.at[idxv.at[s, 1]], rows[b],
                                  gsems[b]).wait()

        def start_scatter(s, b):
            pltpu.async_copy(rows[b], acc.at[idxv.at[s, 0]], ssems[b],
                             add=True)

        def wait_scatter(s, b):
            pltpu.make_async_copy(rows[b], acc.at[idxv.at[s, 0]],
                                  ssems[b]).wait()

        for s in range(5):
            start_idx(s, s)
        wait_idx(0, 0)
        start_gather(0, 0)
        wait_idx(1, 1)
        start_gather(1, 1)

        @pl.loop(0, maxn + 5, step=6)
        def _(j):
            for b in range(6):
                k = j + b
                s6, b3 = b % 6, b % 3

                @pl.when(k < n_w)
                def _():
                    wait_gather(s6, b3)
                    start_scatter(s6, b3)

                @pl.when((k >= 1) & (k - 1 < n_w))
                def _():
                    wait_scatter((b + 5) % 6, (b + 2) % 3)

                @pl.when(k + 2 < n_w)
                def _():
                    wait_idx(k + 2, (b + 2) % 6)
                    start_gather((b + 2) % 6, (b + 2) % 3)

                @pl.when(k + 5 < n_w)
                def _():
                    start_idx(k + 5, (b + 5) % 6)

        plsc.subcore_barrier()

        pltpu.sync_copy(acc.at[pl.ds(sid * 625, 625)],
                        out_h.at[core, pl.ds(sid * 625, 625)])

    kfn = pl.kernel(body, mesh=mesh, scratch_types=scratch,
                    out_type=out_type,
                    compiler_params=pltpu.CompilerParams(
                        use_tc_tiling_on_sc=False))
    return kfn(table, edges, jnp.zeros((ZROWS, d), jnp.float32))


def _tc_matmul_norm_relu(p, degp, w):
    M, K = p.shape[1], p.shape[2]
    N = w.shape[1]
    BM = 1000

    def body(p_ref, d_ref, w_ref, o_ref):
        s = p_ref[0] + p_ref[1]
        m = jnp.dot(s, w_ref[...], preferred_element_type=jnp.float32)
        deg = d_ref[0, :, 0] + d_ref[1, :, 0]
        o_ref[...] = jnp.maximum(m / jnp.maximum(deg, 1.0)[:, None], 0.0)

    return pl.pallas_call(
        body,
        grid=(M // BM,),
        in_specs=[pl.BlockSpec((2, BM, K), lambda i: (0, i, 0)),
                  pl.BlockSpec((2, BM, L), lambda i: (0, i, 0)),
                  pl.BlockSpec((K, N), lambda i: (0, 0))],
        out_specs=pl.BlockSpec((BM, N), lambda i: (i, 0)),
        out_shape=jax.ShapeDtypeStruct((M, N), jnp.float32),
    )(p, degp, w)


def _tc_matmul(x, w):
    M, K = x.shape
    N = w.shape[1]
    BM = 1000

    def body(x_ref, w_ref, o_ref):
        o_ref[...] = jnp.dot(x_ref[...], w_ref[...],
                             preferred_element_type=jnp.float32)

    return pl.pallas_call(
        body,
        grid=(M // BM,),
        in_specs=[pl.BlockSpec((BM, K), lambda i: (i, 0)),
                  pl.BlockSpec((K, N), lambda i: (0, 0))],
        out_specs=pl.BlockSpec((BM, N), lambda i: (i, 0)),
        out_shape=jax.ShapeDtypeStruct((M, N), jnp.float32),
    )(x, w)


def _tc_norm_relu(p, degp):
    M, N = p.shape[1], p.shape[2]
    BM = 1000

    def body(p_ref, d_ref, o_ref):
        s = p_ref[0] + p_ref[1]
        deg = d_ref[0, :, 0] + d_ref[1, :, 0]
        o_ref[...] = jnp.maximum(s / jnp.maximum(deg, 1.0)[:, None], 0.0)

    return pl.pallas_call(
        body,
        grid=(M // BM,),
        in_specs=[pl.BlockSpec((2, BM, N), lambda i: (0, i, 0)),
                  pl.BlockSpec((2, BM, L), lambda i: (0, i, 0))],
        out_specs=pl.BlockSpec((BM, N), lambda i: (i, 0)),
        out_shape=jax.ShapeDtypeStruct((M, N), jnp.float32),
    )(p, degp)


def kernel(x0, x1, edges_s1_01, edges_s1_10, edges_s2_01, edges_s2_10,
           W_s1l1_01, W_s1l1_10, W_s1l2_01, W_s1l2_10,
           W_s2l1_01, W_s2l1_10, W_s2l2_01, W_s2l2_10):
    def chunked(e):
        e = e.astype(jnp.int32).reshape(2, NCHUNKS, CHUNK).transpose(1, 0, 2)
        return jnp.concatenate([e, jnp.zeros((4, 2, CHUNK), jnp.int32)])

    e101 = chunked(edges_s1_01)
    e110 = chunked(edges_s1_10)
    e201 = chunked(edges_s2_01)
    e210 = chunked(edges_s2_10)

    degs = _sc_degrees([e101, e110, e201, e210], N0)
    D101, D110, D201, D210 = degs[0], degs[1], degs[2], degs[3]

    h1_0 = _tc_matmul_norm_relu(_sc_agg(x1, e101, N0), D101, W_s1l1_01)
    h1_1 = _tc_matmul_norm_relu(_sc_agg(x0, e110, N1), D110, W_s1l1_10)

    h2_0 = _tc_norm_relu(_sc_agg(_tc_matmul(h1_1, W_s1l2_01), e101, N0), D101)
    h2_1 = _tc_norm_relu(_sc_agg(_tc_matmul(h1_0, W_s1l2_10), e110, N1), D110)

    h3_0 = _tc_matmul_norm_relu(_sc_agg(h2_1, e201, N0), D201, W_s2l1_01)
    h3_1 = _tc_matmul_norm_relu(_sc_agg(h2_0, e210, N1), D210, W_s2l1_10)

    h4_0 = _tc_norm_relu(_sc_agg(_tc_matmul(h3_1, W_s2l2_01), e201, N0), D201)
    h4_1 = _tc_norm_relu(_sc_agg(_tc_matmul(h3_0, W_s2l2_10), e210, N1), D210)
    return (h4_0, h4_1)


# speedup vs baseline: 1.1517x; 1.1517x over previous
import jax
import jax.numpy as jnp
from jax import lax
from jax.experimental import pallas as pl
from jax.experimental.pallas import tpu as pltpu
from jax.experimental.pallas import tpu_sc as plsc

N0 = 10000
N1 = 10000
E = 320000

NC = 2
NS = 16
NW = NC * NS
L = 16
CHUNK = 128
NCHUNKS = E // CHUNK
MAXC = -(-NCHUNKS // NW)
ZROWS = 125


def _worker_split(wid):
    base_n = NCHUNKS // NW
    cut = NW - (NCHUNKS - base_n * NW)
    n_w = base_n + jnp.where(wid >= cut, 1, 0)
    chunk0 = wid * base_n + jnp.maximum(wid - cut, 0)
    return n_w, chunk0, base_n + 1


def _sc_degrees(edge_sets, n_out):
    mesh = plsc.VectorSubcoreMesh(core_axis_name="c", subcore_axis_name="s")
    nsets = len(edge_sets)
    out_type = jax.ShapeDtypeStruct((nsets, NC, n_out, L), jnp.float32)
    scratch = (
        [pltpu.VMEM((4, 4, 2, CHUNK), jnp.int32),
         pltpu.VMEM((CHUNK, L), jnp.float32),
         pltpu.VMEM((ZROWS, L), jnp.float32)]
        + [pltpu.VMEM_SHARED((n_out, L), jnp.float32) for _ in range(nsets)]
        + [pltpu.SemaphoreType.DMA for _ in range(8)]
    )

    def body(*refs):
        e_hs = refs[:nsets]
        out_h = refs[nsets]
        idxv, onesv, zb = refs[nsets + 1:nsets + 4]
        daccs = refs[nsets + 4:nsets + 4 + nsets]
        isems = refs[nsets + 4 + nsets:nsets + 4 + nsets + 4]
        ssems = refs[nsets + 4 + nsets + 4:]
        core = lax.axis_index("c")
        sid = lax.axis_index("s")
        wid = core * NS + sid
        n_w, chunk0, maxn = _worker_split(wid)

        @pl.loop(0, CHUNK)
        def _(i):
            onesv[i, :] = jnp.ones((L,), jnp.float32)

        @pl.loop(0, ZROWS)
        def _(i):
            zb[i, :] = jnp.zeros((L,), jnp.float32)

        for dacc in daccs:
            for k in range(5):
                pltpu.sync_copy(zb, dacc.at[pl.ds(sid * 625 + k * ZROWS,
                                                  ZROWS)])
        plsc.subcore_barrier()

        for e in range(nsets):
            e_h, dacc = e_hs[e], daccs[e]

            def start_blk(g, s, e_h=e_h):
                pltpu.make_async_copy(e_h.at[pl.ds(chunk0 + 4 * g, 4)],
                                      idxv.at[s], isems[s]).start()

            def wait_blk(g, s, e_h=e_h):
                pltpu.make_async_copy(e_h.at[pl.ds(chunk0 + 4 * g, 4)],
                                      idxv.at[s], isems[s]).wait()

            for s in range(3):
                start_blk(s, s)

            @pl.loop(0, maxn + 17, step=16)
            def _(j):
                for b in range(16):
                    k = j + b
                    blk, win = (b // 4) % 4, b % 4

                    @pl.when(k < n_w)
                    def _():
                        if win == 0:
                            wait_blk(k // 4, blk)
                        pltpu.async_copy(onesv, dacc.at[idxv.at[blk, win, 0]],
                                         ssems[win], add=True)

                    @pl.when((k >= 3) & (k - 3 < n_w))
                    def _():
                        pltpu.make_async_copy(
                            onesv, dacc.at[idxv.at[blk, win, 0]],
                            ssems[(win + 1) % 4]).wait()

                    if win == 2:
                        @pl.when(k + 10 < n_w)
                        def _():
                            start_blk(k // 4 + 3, (blk + 3) % 4)

        plsc.subcore_barrier()

        @pl.when(sid == 0)
        def _():
            for e in range(nsets):
                pltpu.sync_copy(daccs[e], out_h.at[e, core])

    kfn = pl.kernel(body, mesh=mesh, scratch_types=scratch, out_type=out_type,
                    compiler_params=pltpu.CompilerParams(
                        use_tc_tiling_on_sc=False))
    return kfn(*edge_sets)


def _sc_agg(table, edges, n_out):
    d = table.shape[1]
    mesh = plsc.VectorSubcoreMesh(core_axis_name="c", subcore_axis_name="s")

    out_type = jax.ShapeDtypeStruct((NC, n_out, d), jnp.float32)
    scratch = [
        pltpu.VMEM((6, 2, CHUNK), jnp.int32),
        pltpu.VMEM((CHUNK, d), jnp.float32),
        pltpu.VMEM((CHUNK, d), jnp.float32),
        pltpu.VMEM((CHUNK, d), jnp.float32),
        pltpu.VMEM_SHARED((n_out, d), jnp.float32),
    ] + [pltpu.SemaphoreType.DMA] * 12
    def body(table_h, e_h, out_h, idxv, rows0, rows1, rows2, acc, *sems):
        isems, gsems, ssems = sems[:6], sems[6:9], sems[9:12]
        rows = (rows0, rows1, rows2)
        core = lax.axis_index("c")
        sid = lax.axis_index("s")
        wid = core * NS + sid
        n_w, chunk0, maxn = _worker_split(wid)

        @pl.loop(0, ZROWS)
        def _(i):
            @pl.loop(0, d // L)
            def _(c):
                rows0[i, pl.ds(c * L, L)] = jnp.zeros((L,), jnp.float32)

        for k in range(5):
            pltpu.sync_copy(rows0.at[pl.ds(0, ZROWS)],
                            acc.at[pl.ds(sid * 625 + k * ZROWS, ZROWS)])
        plsc.subcore_barrier()

        def start_idx(k, s):
            pltpu.make_async_copy(e_h.at[chunk0 + k], idxv.at[s],
                                  isems[s]).start()

        def wait_idx(k, s):
            pltpu.make_async_copy(e_h.at[chunk0 + k], idxv.at[s],
                                  isems[s]).wait()

        def start_gather(s, b):
            pltpu.make_async_copy(table_h.at[idxv.at[s, 1]], rows[b],
                                  gsems[b]).start()

        def wait_gather(s, b):
            pltpu.make_async_copy(table_h.at[idxv.at[s, 1]], rows[b],
                                  gsems[b]).wait()

        def start_scatter(s, b):
            pltpu.async_copy(rows[b], acc.at[idxv.at[s, 0]], ssems[b],
                             add=True)

        def wait_scatter(s, b):
            pltpu.make_async_copy(rows[b], acc.at[idxv.at[s, 0]],
                                  ssems[b]).wait()

        for s in range(5):
            start_idx(s, s)
        wait_idx(0, 0)
        start_gather(0, 0)
        wait_idx(1, 1)
        start_gather(1, 1)

        @pl.loop(0, maxn + 5, step=6)
        def _(j):
            for b in range(6):
                k = j + b
                s6, b3 = b % 6, b % 3

                @pl.when(k < n_w)
                def _():
                    wait_gather(s6, b3)
                    start_scatter(s6, b3)

                @pl.when((k >= 1) & (k - 1 < n_w))
                def _():
                    wait_scatter((b + 5) % 6, (b + 2) % 3)

                @pl.when(k + 2 < n_w)
                def _():
                    wait_idx(k + 2, (b + 2) % 6)
                    start_gather((b + 2) % 6, (b + 2) % 3)

                @pl.when(k + 5 < n_w)
                def _():
                    start_idx(k + 5, (b + 5) % 6)

        plsc.subcore_barrier()

        @pl.when(sid == 0)
        def _():
            pltpu.sync_copy(acc, out_h.at[core])

    kfn = pl.kernel(body, mesh=mesh, scratch_types=scratch,
                    out_type=out_type,
                    compiler_params=pltpu.CompilerParams(
                        use_tc_tiling_on_sc=False))
    return kfn(table, edges)


def _tc_matmul_norm_relu(p, degp, w):
    M, K = p.shape[1], p.shape[2]
    N = w.shape[1]
    BM = 1000

    def body(p_ref, d_ref, w_ref, o_ref):
        s = p_ref[0] + p_ref[1]
        m = jnp.dot(s, w_ref[...], preferred_element_type=jnp.float32)
        deg = d_ref[0, :, 0] + d_ref[1, :, 0]
        o_ref[...] = jnp.maximum(m / jnp.maximum(deg, 1.0)[:, None], 0.0)

    return pl.pallas_call(
        body,
        grid=(M // BM,),
        in_specs=[pl.BlockSpec((2, BM, K), lambda i: (0, i, 0)),
                  pl.BlockSpec((2, BM, L), lambda i: (0, i, 0)),
                  pl.BlockSpec((K, N), lambda i: (0, 0))],
        out_specs=pl.BlockSpec((BM, N), lambda i: (i, 0)),
        out_shape=jax.ShapeDtypeStruct((M, N), jnp.float32),
    )(p, degp, w)


def _tc_matmul(x, w):
    M, K = x.shape
    N = w.shape[1]
    BM = 1000

    def body(x_ref, w_ref, o_ref):
        o_ref[...] = jnp.dot(x_ref[...], w_ref[...],
                             preferred_element_type=jnp.float32)

    return pl.pallas_call(
        body,
        grid=(M // BM,),
        in_specs=[pl.BlockSpec((BM, K), lambda i: (i, 0)),
                  pl.BlockSpec((K, N), lambda i: (0, 0))],
        out_specs=pl.BlockSpec((BM, N), lambda i: (i, 0)),
        out_shape=jax.ShapeDtypeStruct((M, N), jnp.float32),
    )(x, w)


def _tc_norm_relu(p, degp):
    M, N = p.shape[1], p.shape[2]
    BM = 1000

    def body(p_ref, d_ref, o_ref):
        s = p_ref[0] + p_ref[1]
        deg = d_ref[0, :, 0] + d_ref[1, :, 0]
        o_ref[...] = jnp.maximum(s / jnp.maximum(deg, 1.0)[:, None], 0.0)

    return pl.pallas_call(
        body,
        grid=(M // BM,),
        in_specs=[pl.BlockSpec((2, BM, N), lambda i: (0, i, 0)),
                  pl.BlockSpec((2, BM, L), lambda i: (0, i, 0))],
        out_specs=pl.BlockSpec((BM, N), lambda i: (i, 0)),
        out_shape=jax.ShapeDtypeStruct((M, N), jnp.float32),
    )(p, degp)


def kernel(x0, x1, edges_s1_01, edges_s1_10, edges_s2_01, edges_s2_10,
           W_s1l1_01, W_s1l1_10, W_s1l2_01, W_s1l2_10,
           W_s2l1_01, W_s2l1_10, W_s2l2_01, W_s2l2_10):
    def chunked(e):
        e = e.astype(jnp.int32).reshape(2, NCHUNKS, CHUNK).transpose(1, 0, 2)
        return jnp.concatenate([e, jnp.zeros((4, 2, CHUNK), jnp.int32)])

    e101 = chunked(edges_s1_01)
    e110 = chunked(edges_s1_10)
    e201 = chunked(edges_s2_01)
    e210 = chunked(edges_s2_10)

    degs = _sc_degrees([e101, e110, e201, e210], N0)
    D101, D110, D201, D210 = degs[0], degs[1], degs[2], degs[3]

    h1_0 = _tc_matmul_norm_relu(_sc_agg(x1, e101, N0), D101, W_s1l1_01)
    h1_1 = _tc_matmul_norm_relu(_sc_agg(x0, e110, N1), D110, W_s1l1_10)

    h2_0 = _tc_norm_relu(_sc_agg(_tc_matmul(h1_1, W_s1l2_01), e101, N0), D101)
    h2_1 = _tc_norm_relu(_sc_agg(_tc_matmul(h1_0, W_s1l2_10), e110, N1), D110)

    h3_0 = _tc_matmul_norm_relu(_sc_agg(h2_1, e201, N0), D201, W_s2l1_01)
    h3_1 = _tc_matmul_norm_relu(_sc_agg(h2_0, e210, N1), D210, W_s2l1_10)

    h4_0 = _tc_norm_relu(_sc_agg(_tc_matmul(h3_1, W_s2l2_01), e201, N0), D201)
    h4_1 = _tc_norm_relu(_sc_agg(_tc_matmul(h3_0, W_s2l2_10), e210, N1), D210)
    return (h4_0, h4_1)


# speedup vs baseline: 1.1972x; 1.0395x over previous
import jax
import jax.numpy as jnp
from jax import lax
from jax.experimental import pallas as pl
from jax.experimental.pallas import tpu as pltpu
from jax.experimental.pallas import tpu_sc as plsc

N0 = 10000
N1 = 10000
E = 320000

NC = 2
NS = 16
NW = NC * NS
L = 16
CHUNK = 128
NCHUNKS = E // CHUNK
MAXC = -(-NCHUNKS // NW)
ZROWS = 125


def _worker_split(wid):
    base_n = NCHUNKS // NW
    cut = NW - (NCHUNKS - base_n * NW)
    n_w = base_n + jnp.where(wid >= cut, 1, 0)
    chunk0 = wid * base_n + jnp.maximum(wid - cut, 0)
    return n_w, chunk0, base_n + 1


def _sc_degrees(edge_sets, n_out):
    mesh = plsc.VectorSubcoreMesh(core_axis_name="c", subcore_axis_name="s")
    nsets = len(edge_sets)
    nrow = n_out // L
    nrp = 640
    out_type = jax.ShapeDtypeStruct((nsets, NC, nrp, L), jnp.float32)
    scratch = (
        [pltpu.VMEM((6, 4, 2, CHUNK), jnp.int32),
         pltpu.VMEM((CHUNK, L), jnp.float32),
         pltpu.VMEM((5, CHUNK), jnp.int32)]
        + [pltpu.VMEM((nrp, L), jnp.float32) for _ in range(nsets)]
        + [pltpu.VMEM_SHARED((nrp, L), jnp.float32) for _ in range(nsets)]
        + [pltpu.SemaphoreType.DMA for _ in range(6)]
    )

    def body(*refs):
        e_hs = refs[:nsets]
        out_h = refs[nsets]
        idxv, zb, mrg = refs[nsets + 1:nsets + 4]
        degs = refs[nsets + 4:nsets + 4 + nsets]
        daccs = refs[nsets + 4 + nsets:nsets + 4 + 2 * nsets]
        isems = refs[nsets + 4 + 2 * nsets:]
        core = lax.axis_index("c")
        sid = lax.axis_index("s")
        wid = core * NS + sid
        n_w, chunk0, maxn = _worker_split(wid)
        ones16 = jnp.ones((L,), jnp.float32)

        @pl.loop(0, CHUNK)
        def _(i):
            zb[i, :] = jnp.zeros((L,), jnp.float32)

        @pl.loop(0, 5)
        def _(t):
            @pl.loop(0, CHUNK // L)
            def _(g):
                mrg[t, pl.ds(g * L, L)] = (lax.iota(jnp.int32, L)
                                           + t * CHUNK + g * L)

        for e in range(nsets):
            deg = degs[e]

            @pl.loop(0, nrp)
            def _(i, deg=deg):
                deg[i, :] = jnp.zeros((L,), jnp.float32)

            pltpu.sync_copy(zb.at[pl.ds(0, nrp // NS)],
                            daccs[e].at[pl.ds(sid * (nrp // NS), nrp // NS)])
        plsc.subcore_barrier()

        for e in range(nsets):
            e_h, deg = e_hs[e], degs[e]

            def start_blk(g, s, e_h=e_h):
                pltpu.make_async_copy(e_h.at[pl.ds(chunk0 + 4 * g, 4)],
                                      idxv.at[s], isems[s]).start()

            def wait_blk(g, s, e_h=e_h):
                pltpu.make_async_copy(e_h.at[pl.ds(chunk0 + 4 * g, 4)],
                                      idxv.at[s], isems[s]).wait()

            for s in range(6):
                start_blk(s, s)

            @pl.loop(0, 24, step=6)
            def _(j):
                for b in range(6):
                    g = j + b

                    @pl.when(4 * g < n_w)
                    def _():
                        wait_blk(g, b)
                        for w in range(4):
                            @pl.when(4 * g + w < n_w)
                            def _():
                                for gi in range(CHUNK // L):
                                    idx = idxv[b, w, 0, pl.ds(gi * L, L)]
                                    hi = lax.shift_right_logical(idx, 4)
                                    lo = jnp.bitwise_and(idx, 15)
                                    plsc.addupdate_scatter(deg, [hi, lo],
                                                           ones16)

                    @pl.when(4 * (g + 6) < n_w)
                    def _():
                        start_blk(g + 6, b)

            for t in range(5):
                pltpu.sync_copy(deg.at[pl.ds(t * CHUNK, CHUNK)],
                                daccs[e].at[mrg.at[t]], add=True)

        plsc.subcore_barrier()

        @pl.when(sid == 0)
        def _():
            for e in range(nsets):
                pltpu.sync_copy(daccs[e], out_h.at[e, core])

    kfn = pl.kernel(body, mesh=mesh, scratch_types=scratch, out_type=out_type,
                    compiler_params=pltpu.CompilerParams(
                        use_tc_tiling_on_sc=False,
                        needs_layout_passes=False))
    return kfn(*edge_sets)


def _sc_agg(table, edges, n_out):
    d = table.shape[1]
    mesh = plsc.VectorSubcoreMesh(core_axis_name="c", subcore_axis_name="s")

    out_type = jax.ShapeDtypeStruct((NC, n_out, d), jnp.float32)
    scratch = [
        pltpu.VMEM((6, 2, CHUNK), jnp.int32),
        pltpu.VMEM((CHUNK, d), jnp.float32),
        pltpu.VMEM((CHUNK, d), jnp.float32),
        pltpu.VMEM((CHUNK, d), jnp.float32),
        pltpu.VMEM_SHARED((n_out, d), jnp.float32),
    ] + [pltpu.SemaphoreType.DMA] * 12
    def body(table_h, e_h, out_h, idxv, rows0, rows1, rows2, acc, *sems):
        isems, gsems, ssems = sems[:6], sems[6:9], sems[9:12]
        rows = (rows0, rows1, rows2)
        core = lax.axis_index("c")
        sid = lax.axis_index("s")
        wid = core * NS + sid
        n_w, chunk0, maxn = _worker_split(wid)

        @pl.loop(0, ZROWS)
        def _(i):
            @pl.loop(0, d // L)
            def _(c):
                rows0[i, pl.ds(c * L, L)] = jnp.zeros((L,), jnp.float32)

        for k in range(5):
            pltpu.sync_copy(rows0.at[pl.ds(0, ZROWS)],
                            acc.at[pl.ds(sid * 625 + k * ZROWS, ZROWS)])
        plsc.subcore_barrier()

        def start_idx(k, s):
            pltpu.make_async_copy(e_h.at[chunk0 + k], idxv.at[s],
                                  isems[s]).start()

        def wait_idx(k, s):
            pltpu.make_async_copy(e_h.at[chunk0 + k], idxv.at[s],
                                  isems[s]).wait()

        def start_gather(s, b):
            pltpu.make_async_copy(table_h.at[idxv.at[s, 1]], rows[b],
                                  gsems[b]).start()

        def wait_gather(s, b):
            pltpu.make_async_copy(table_h.at[idxv.at[s, 1]], rows[b],
                                  gsems[b]).wait()

        def start_scatter(s, b):
            pltpu.async_copy(rows[b], acc.at[idxv.at[s, 0]], ssems[b],
                             add=True)

        def wait_scatter(s, b):
            pltpu.make_async_copy(rows[b], acc.at[idxv.at[s, 0]],
                                  ssems[b]).wait()

        for s in range(5):
            start_idx(s, s)
        wait_idx(0, 0)
        start_gather(0, 0)
        wait_idx(1, 1)
        start_gather(1, 1)

        @pl.loop(0, maxn + 5, step=6)
        def _(j):
            for b in range(6):
                k = j + b
                s6, b3 = b % 6, b % 3

                @pl.when(k < n_w)
                def _():
                    wait_gather(s6, b3)
                    start_scatter(s6, b3)

                @pl.when((k >= 1) & (k - 1 < n_w))
                def _():
                    wait_scatter((b + 5) % 6, (b + 2) % 3)

                @pl.when(k + 2 < n_w)
                def _():
                    wait_idx(k + 2, (b + 2) % 6)
                    start_gather((b + 2) % 6, (b + 2) % 3)

                @pl.when(k + 5 < n_w)
                def _():
                    start_idx(k + 5, (b + 5) % 6)

        plsc.subcore_barrier()

        @pl.when(sid == 0)
        def _():
            pltpu.sync_copy(acc, out_h.at[core])

    kfn = pl.kernel(body, mesh=mesh, scratch_types=scratch,
                    out_type=out_type,
                    compiler_params=pltpu.CompilerParams(
                        use_tc_tiling_on_sc=False))
    return kfn(table, edges)


def _tc_matmul_norm_relu(p, degp, w):
    M, K = p.shape[1], p.shape[2]
    N = w.shape[1]
    BM = 1000

    def body(p_ref, d_ref, w_ref, o_ref):
        s = p_ref[0] + p_ref[1]
        m = jnp.dot(s, w_ref[...], preferred_element_type=jnp.float32)
        deg = d_ref[0, 0] + d_ref[0, 1]
        o_ref[...] = jnp.maximum(m / jnp.maximum(deg, 1.0)[:, None], 0.0)

    return pl.pallas_call(
        body,
        grid=(M // BM,),
        in_specs=[pl.BlockSpec((2, BM, K), lambda i: (0, i, 0)),
                  pl.BlockSpec((1, 2, BM), lambda i: (i, 0, 0)),
                  pl.BlockSpec((K, N), lambda i: (0, 0))],
        out_specs=pl.BlockSpec((BM, N), lambda i: (i, 0)),
        out_shape=jax.ShapeDtypeStruct((M, N), jnp.float32),
    )(p, degp, w)


def _tc_matmul(x, w):
    M, K = x.shape
    N = w.shape[1]
    BM = 1000

    def body(x_ref, w_ref, o_ref):
        o_ref[...] = jnp.dot(x_ref[...], w_ref[...],
                             preferred_element_type=jnp.float32)

    return pl.pallas_call(
        body,
        grid=(M // BM,),
        in_specs=[pl.BlockSpec((BM, K), lambda i: (i, 0)),
                  pl.BlockSpec((K, N), lambda i: (0, 0))],
        out_specs=pl.BlockSpec((BM, N), lambda i: (i, 0)),
        out_shape=jax.ShapeDtypeStruct((M, N), jnp.float32),
    )(x, w)


def _tc_norm_relu(p, degp):
    M, N = p.shape[1], p.shape[2]
    BM = 1000

    def body(p_ref, d_ref, o_ref):
        s = p_ref[0] + p_ref[1]
        deg = d_ref[0, 0] + d_ref[0, 1]
        o_ref[...] = jnp.maximum(s / jnp.maximum(deg, 1.0)[:, None], 0.0)

    return pl.pallas_call(
        body,
        grid=(M // BM,),
        in_specs=[pl.BlockSpec((2, BM, N), lambda i: (0, i, 0)),
                  pl.BlockSpec((1, 2, BM), lambda i: (i, 0, 0))],
        out_specs=pl.BlockSpec((BM, N), lambda i: (i, 0)),
        out_shape=jax.ShapeDtypeStruct((M, N), jnp.float32),
    )(p, degp)


def kernel(x0, x1, edges_s1_01, edges_s1_10, edges_s2_01, edges_s2_10,
           W_s1l1_01, W_s1l1_10, W_s1l2_01, W_s1l2_10,
           W_s2l1_01, W_s2l1_10, W_s2l2_01, W_s2l2_10):
    def chunked(e):
        e = e.astype(jnp.int32).reshape(2, NCHUNKS, CHUNK).transpose(1, 0, 2)
        return jnp.concatenate([e, jnp.zeros((4, 2, CHUNK), jnp.int32)])

    e101 = chunked(edges_s1_01)
    e110 = chunked(edges_s1_10)
    e201 = chunked(edges_s2_01)
    e210 = chunked(edges_s2_10)

    degs = _sc_degrees([e101, e110, e201, e210], N0)
    degs = degs.reshape(4, NC, -1)[:, :, :N0]
    degs = degs.reshape(4, NC, 10, 1000).transpose(0, 2, 1, 3)
    D101, D110, D201, D210 = degs[0], degs[1], degs[2], degs[3]

    h1_0 = _tc_matmul_norm_relu(_sc_agg(x1, e101, N0), D101, W_s1l1_01)
    h1_1 = _tc_matmul_norm_relu(_sc_agg(x0, e110, N1), D110, W_s1l1_10)

    h2_0 = _tc_norm_relu(_sc_agg(_tc_matmul(h1_1, W_s1l2_01), e101, N0), D101)
    h2_1 = _tc_norm_relu(_sc_agg(_tc_matmul(h1_0, W_s1l2_10), e110, N1), D110)

    h3_0 = _tc_matmul_norm_relu(_sc_agg(h2_1, e201, N0), D201, W_s2l1_01)
    h3_1 = _tc_matmul_norm_relu(_sc_agg(h2_0, e210, N1), D210, W_s2l1_10)

    h4_0 = _tc_norm_relu(_sc_agg(_tc_matmul(h3_1, W_s2l2_01), e201, N0), D201)
    h4_1 = _tc_norm_relu(_sc_agg(_tc_matmul(h3_0, W_s2l2_10), e210, N1), D210)
    return (h4_0, h4_1)


# speedup vs baseline: 1.1979x; 1.0006x over previous
import jax
import jax.numpy as jnp
from jax import lax
from jax.experimental import pallas as pl
from jax.experimental.pallas import tpu as pltpu
from jax.experimental.pallas import tpu_sc as plsc

N0 = 10000
N1 = 10000
E = 320000

NC = 2
NS = 16
NW = NC * NS
L = 16
CHUNK = 128
NCHUNKS = E // CHUNK
ZROWS = 125


def _worker_split(wid):
    base_n = NCHUNKS // NW
    cut = NW - (NCHUNKS - base_n * NW)
    n_w = base_n + jnp.where(wid >= cut, 1, 0)
    chunk0 = wid * base_n + jnp.maximum(wid - cut, 0)
    return n_w, chunk0, base_n + 1


def _sc_degrees(edge_sets, n_out):
    mesh = plsc.VectorSubcoreMesh(core_axis_name="c", subcore_axis_name="s")
    nsets = len(edge_sets)
    nrp = 640
    out_type = jax.ShapeDtypeStruct((nsets, NC, nrp, L), jnp.float32)
    scratch = (
        [pltpu.VMEM((6, 4, 2, CHUNK), jnp.int32),
         pltpu.VMEM((CHUNK, L), jnp.float32),
         pltpu.VMEM((5, CHUNK), jnp.int32)]
        + [pltpu.VMEM((nrp, L), jnp.float32) for _ in range(nsets)]
        + [pltpu.VMEM_SHARED((nrp, L), jnp.float32) for _ in range(nsets)]
        + [pltpu.SemaphoreType.DMA for _ in range(6)]
    )

    def body(*refs):
        e_hs = refs[:nsets]
        out_h = refs[nsets]
        idxv, zb, mrg = refs[nsets + 1:nsets + 4]
        degs = refs[nsets + 4:nsets + 4 + nsets]
        daccs = refs[nsets + 4 + nsets:nsets + 4 + 2 * nsets]
        isems = refs[nsets + 4 + 2 * nsets:]
        core = lax.axis_index("c")
        sid = lax.axis_index("s")
        wid = core * NS + sid
        n_w, chunk0, maxn = _worker_split(wid)
        ones16 = jnp.ones((L,), jnp.float32)

        @pl.loop(0, CHUNK)
        def _(i):
            zb[i, :] = jnp.zeros((L,), jnp.float32)

        @pl.loop(0, 5)
        def _(t):
            @pl.loop(0, CHUNK // L)
            def _(g):
                mrg[t, pl.ds(g * L, L)] = (lax.iota(jnp.int32, L)
                                           + t * CHUNK + g * L)

        for e in range(nsets):
            deg = degs[e]

            @pl.loop(0, nrp)
            def _(i, deg=deg):
                deg[i, :] = jnp.zeros((L,), jnp.float32)

            pltpu.sync_copy(zb.at[pl.ds(0, nrp // NS)],
                            daccs[e].at[pl.ds(sid * (nrp // NS), nrp // NS)])
        plsc.subcore_barrier()

        for e in range(nsets):
            e_h, deg = e_hs[e], degs[e]

            def start_blk(g, s, e_h=e_h):
                pltpu.make_async_copy(e_h.at[pl.ds(chunk0 + 4 * g, 4)],
                                      idxv.at[s], isems[s]).start()

            def wait_blk(g, s, e_h=e_h):
                pltpu.make_async_copy(e_h.at[pl.ds(chunk0 + 4 * g, 4)],
                                      idxv.at[s], isems[s]).wait()

            for s in range(6):
                start_blk(s, s)

            @pl.loop(0, 24, step=6)
            def _(j):
                for b in range(6):
                    g = j + b

                    @pl.when(4 * g < n_w)
                    def _():
                        wait_blk(g, b)
                        for w in range(4):
                            @pl.when(4 * g + w < n_w)
                            def _():
                                for gi in range(CHUNK // L):
                                    idx = idxv[b, w, 0, pl.ds(gi * L, L)]
                                    hi = lax.shift_right_logical(idx, 4)
                                    lo = jnp.bitwise_and(idx, 15)
                                    plsc.addupdate_scatter(deg, [hi, lo],
                                                           ones16)

                    @pl.when(4 * (g + 6) < n_w)
                    def _():
                        start_blk(g + 6, b)

            for t in range(5):
                pltpu.sync_copy(deg.at[pl.ds(t * CHUNK, CHUNK)],
                                daccs[e].at[mrg.at[t]], add=True)

        plsc.subcore_barrier()

        @pl.when(sid == 0)
        def _():
            for e in range(nsets):
                pltpu.sync_copy(daccs[e], out_h.at[e, core])

    kfn = pl.kernel(body, mesh=mesh, scratch_types=scratch, out_type=out_type,
                    compiler_params=pltpu.CompilerParams(
                        use_tc_tiling_on_sc=False,
                        needs_layout_passes=False))
    return kfn(*edge_sets)


def _sc_agg(table, edges, n_out):
    d = table.shape[1]
    mesh = plsc.VectorSubcoreMesh(core_axis_name="c", subcore_axis_name="s")

    out_type = jax.ShapeDtypeStruct((NC, n_out, d), jnp.float32)
    scratch = [
        pltpu.VMEM((6, 2, CHUNK), jnp.int32),
        pltpu.VMEM((CHUNK, d), jnp.float32),
        pltpu.VMEM((CHUNK, d), jnp.float32),
        pltpu.VMEM((CHUNK, d), jnp.float32),
        pltpu.VMEM_SHARED((n_out, d), jnp.float32),
    ] + [pltpu.SemaphoreType.DMA] * 12
    def body(table_h, e_h, out_h, idxv, rows0, rows1, rows2, acc, *sems):
        isems, gsems, ssems = sems[:6], sems[6:9], sems[9:12]
        rows = (rows0, rows1, rows2)
        core = lax.axis_index("c")
        sid = lax.axis_index("s")
        wid = core * NS + sid
        n_w, chunk0, maxn = _worker_split(wid)

        @pl.loop(0, ZROWS)
        def _(i):
            @pl.loop(0, d // L)
            def _(c):
                rows0[i, pl.ds(c * L, L)] = jnp.zeros((L,), jnp.float32)

        for k in range(5):
            pltpu.sync_copy(rows0.at[pl.ds(0, ZROWS)],
                            acc.at[pl.ds(sid * 625 + k * ZROWS, ZROWS)])
        plsc.subcore_barrier()

        def start_idx(k, s):
            pltpu.make_async_copy(e_h.at[chunk0 + k], idxv.at[s],
                                  isems[s]).start()

        def wait_idx(k, s):
            pltpu.make_async_copy(e_h.at[chunk0 + k], idxv.at[s],
                                  isems[s]).wait()

        def start_gather(s, b):
            pltpu.make_async_copy(table_h.at[idxv.at[s, 1]], rows[b],
                                  gsems[b]).start()

        def wait_gather(s, b):
            pltpu.make_async_copy(table_h.at[idxv.at[s, 1]], rows[b],
                                  gsems[b]).wait()

        def start_scatter(s, b):
            pltpu.async_copy(rows[b], acc.at[idxv.at[s, 0]], ssems[b],
                             add=True)

        def wait_scatter(s, b):
            pltpu.make_async_copy(rows[b], acc.at[idxv.at[s, 0]],
                                  ssems[b]).wait()

        for s in range(5):
            start_idx(s, s)
        wait_idx(0, 0)
        start_gather(0, 0)
        wait_idx(1, 1)
        start_gather(1, 1)

        @pl.loop(0, maxn + 5, step=6)
        def _(j):
            for b in range(6):
                k = j + b
                s6, b3 = b % 6, b % 3

                @pl.when(k < n_w)
                def _():
                    wait_gather(s6, b3)
                    start_scatter(s6, b3)

                @pl.when((k >= 1) & (k - 1 < n_w))
                def _():
                    wait_scatter((b + 5) % 6, (b + 2) % 3)

                @pl.when(k + 2 < n_w)
                def _():
                    wait_idx(k + 2, (b + 2) % 6)
                    start_gather((b + 2) % 6, (b + 2) % 3)

                @pl.when(k + 5 < n_w)
                def _():
                    start_idx(k + 5, (b + 5) % 6)

        plsc.subcore_barrier()

        @pl.when(sid == 0)
        def _():
            pltpu.sync_copy(acc, out_h.at[core])

    kfn = pl.kernel(body, mesh=mesh, scratch_types=scratch,
                    out_type=out_type,
                    compiler_params=pltpu.CompilerParams(
                        use_tc_tiling_on_sc=False))
    return kfn(table, edges)


def _tc_matmul_norm_relu(p, degp, w):
    M, K = p.shape[1], p.shape[2]
    N = w.shape[1]
    BM = 1000

    def body(p_ref, d_ref, w_ref, o_ref):
        s = p_ref[0] + p_ref[1]
        m = jnp.dot(s, w_ref[...], preferred_element_type=jnp.float32)
        deg = d_ref[0, 0] + d_ref[0, 1]
        o_ref[...] = jnp.maximum(m / jnp.maximum(deg, 1.0)[:, None], 0.0)

    return pl.pallas_call(
        body,
        grid=(M // BM,),
        in_specs=[pl.BlockSpec((2, BM, K), lambda i: (0, i, 0)),
                  pl.BlockSpec((1, 2, BM), lambda i: (i, 0, 0)),
                  pl.BlockSpec((K, N), lambda i: (0, 0))],
        out_specs=pl.BlockSpec((BM, N), lambda i: (i, 0)),
        out_shape=jax.ShapeDtypeStruct((M, N), jnp.float32),
    )(p, degp, w)


def _tc_matmul(x, w):
    M, K = x.shape
    N = w.shape[1]
    BM = 1000

    def body(x_ref, w_ref, o_ref):
        o_ref[...] = jnp.dot(x_ref[...], w_ref[...],
                             preferred_element_type=jnp.float32)

    return pl.pallas_call(
        body,
        grid=(M // BM,),
        in_specs=[pl.BlockSpec((BM, K), lambda i: (i, 0)),
                  pl.BlockSpec((K, N), lambda i: (0, 0))],
        out_specs=pl.BlockSpec((BM, N), lambda i: (i, 0)),
        out_shape=jax.ShapeDtypeStruct((M, N), jnp.float32),
    )(x, w)


def _tc_norm_relu(p, degp):
    M, N = p.shape[1], p.shape[2]
    BM = 1000

    def body(p_ref, d_ref, o_ref):
        s = p_ref[0] + p_ref[1]
        deg = d_ref[0, 0] + d_ref[0, 1]
        o_ref[...] = jnp.maximum(s / jnp.maximum(deg, 1.0)[:, None], 0.0)

    return pl.pallas_call(
        body,
        grid=(M // BM,),
        in_specs=[pl.BlockSpec((2, BM, N), lambda i: (0, i, 0)),
                  pl.BlockSpec((1, 2, BM), lambda i: (i, 0, 0))],
        out_specs=pl.BlockSpec((BM, N), lambda i: (i, 0)),
        out_shape=jax.ShapeDtypeStruct((M, N), jnp.float32),
    )(p, degp)


def kernel(x0, x1, edges_s1_01, edges_s1_10, edges_s2_01, edges_s2_10,
           W_s1l1_01, W_s1l1_10, W_s1l2_01, W_s1l2_10,
           W_s2l1_01, W_s2l1_10, W_s2l2_01, W_s2l2_10):
    def chunked(e):
        e = e.astype(jnp.int32).reshape(2, NCHUNKS, CHUNK).transpose(1, 0, 2)
        return jnp.concatenate([e, jnp.zeros((4, 2, CHUNK), jnp.int32)])

    e101 = chunked(edges_s1_01)
    e110 = chunked(edges_s1_10)
    e201 = chunked(edges_s2_01)
    e210 = chunked(edges_s2_10)

    degs = _sc_degrees([e101, e110, e201, e210], N0)
    degs = degs.reshape(4, NC, -1)[:, :, :N0]
    degs = degs.reshape(4, NC, 10, 1000).transpose(0, 2, 1, 3)
    D101, D110, D201, D210 = degs[0], degs[1], degs[2], degs[3]

    h1_0 = _tc_matmul_norm_relu(_sc_agg(x1, e101, N0), D101, W_s1l1_01)
    h1_1 = _tc_matmul_norm_relu(_sc_agg(x0, e110, N1), D110, W_s1l1_10)

    h2_0 = _tc_norm_relu(_sc_agg(_tc_matmul(h1_1, W_s1l2_01), e101, N0), D101)
    h2_1 = _tc_norm_relu(_sc_agg(_tc_matmul(h1_0, W_s1l2_10), e110, N1), D110)

    h3_0 = _tc_matmul_norm_relu(_sc_agg(h2_1, e201, N0), D201, W_s2l1_01)
    h3_1 = _tc_matmul_norm_relu(_sc_agg(h2_0, e210, N1), D210, W_s2l1_10)

    h4_0 = _tc_norm_relu(_sc_agg(_tc_matmul(h3_1, W_s2l2_01), e201, N0), D201)
    h4_1 = _tc_norm_relu(_sc_agg(_tc_matmul(h3_0, W_s2l2_10), e210, N1), D210)
    return (h4_0, h4_1)
